# baseline (speedup 1.0000x reference)
import functools

import jax
import jax.numpy as jnp
from jax import lax
from jax.experimental import pallas as pl
from jax.experimental.pallas import tpu as pltpu
from jax.experimental.pallas import tpu_sc as plsc

_D = 64
_P = 50
_H = 2
_BATCH = 4096
_NC = 2
_NS = 16
_NW = _NC * _NS
_NB = _BATCH // _NW
_PJ = _P * _H


def _gather_body(idx_hbm, table_hbm, out_hbm, idx_v, rows_v, outb_v, gsem):
    wid = lax.axis_index("s") * _NC + lax.axis_index("c")
    b0 = wid * _NB
    pltpu.sync_copy(idx_hbm.at[:, :, pl.ds(b0, _NB)], idx_v)

    iota = lax.iota(jnp.int32, 16)

    def chunk(c, carry):
        p = c // _H
        j = c - p * _H
        pltpu.async_copy(
            table_hbm.at[idx_v.at[p, j]], rows_v, gsem
        ).wait()

        def drow(d, carry2):
            dvec = jnp.full((16,), d, jnp.int32)
            for bb in range(_NB // 16):
                vals = plsc.load_gather(rows_v, [iota + (bb * 16), dvec])
                outb_v.at[d][pl.ds(bb * 16, 16)] = vals
            return carry2

        lax.fori_loop(0, _D, drow, 0, unroll=4)
        pltpu.sync_copy(outb_v, out_hbm.at[p, j, :, pl.ds(b0, _NB)])
        return carry

    lax.fori_loop(0, _PJ, chunk, 0)


@jax.jit
def _embed_lookup(idx_t, table):
    mesh = plsc.VectorSubcoreMesh(core_axis_name="c", subcore_axis_name="s")
    run = pl.kernel(
        _gather_body,
        out_type=jax.ShapeDtypeStruct((_P, _H, _D, _BATCH), jnp.float32),
        mesh=mesh,
        scratch_types=[
            pltpu.VMEM((_P, _H, _NB), jnp.int32),
            pltpu.VMEM((_NB, _D), jnp.float32),
            pltpu.VMEM((_D, _NB), jnp.float32),
            pltpu.SemaphoreType.DMA,
        ],
        compiler_params=pltpu.CompilerParams(
            use_tc_tiling_on_sc=False, needs_layout_passes=False
        ),
    )
    return run(idx_t, table)


def kernel(idx, embedding_weight):
    idx_t = jnp.transpose(idx, (1, 2, 0))
    out_t = _embed_lookup(idx_t, embedding_weight)
    return jnp.transpose(out_t, (3, 0, 1, 2))


# speedup vs baseline: 1.7935x; 1.7935x over previous
import functools

import jax
import jax.numpy as jnp
from jax import lax
from jax.experimental import pallas as pl
from jax.experimental.pallas import tpu as pltpu
from jax.experimental.pallas import tpu_sc as plsc

_D = 64
_P = 50
_H = 2
_BATCH = 4096
_NC = 2
_NS = 16
_NW = _NC * _NS
_NB = _BATCH // _NW
_PJ = _P * _H


def _gather_body(idx_hbm, table_hbm, out_hbm, idx_v, rows_v, outb_v, gsem, wsem):
    wid = lax.axis_index("s") * _NC + lax.axis_index("c")
    pltpu.sync_copy(idx_hbm.at[:, wid], idx_v)

    iota = lax.iota(jnp.int32, 16)
    perms = [(iota + k) & 15 for k in range(16)]

    def start_gather(c, s):
        p = c // _H
        j = c - p * _H
        return pltpu.async_copy(
            table_hbm.at[idx_v.at[p, j]], rows_v.at[s], gsem.at[s]
        )

    def start_write(c, s):
        p = c // _H
        j = c - p * _H
        return pltpu.async_copy(
            outb_v.at[s], out_hbm.at[p, j, :, wid], wsem.at[s]
        )

    start_gather(0, 0)

    def chunk(c, carry):
        s = c & 1

        @pl.when(c >= 2)
        def _():
            start_write_desc = pltpu.make_async_copy(
                outb_v.at[s], out_hbm.at[0, 0, :, wid], wsem.at[s]
            )
            start_write_desc.wait()

        @pl.when(c + 1 < _PJ)
        def _():
            start_gather(c + 1, 1 - s)

        pltpu.make_async_copy(
            table_hbm.at[idx_v.at[0, 0]], rows_v.at[s], gsem.at[s]
        ).wait()

        def dblock(t, carry2):
            d0 = t * 16
            dt0 = t * 2
            for bb in range(_NB // 16):
                bv = iota + (bb * 16)
                for k in range(16):
                    pk = perms[k]
                    dv = pk + d0
                    vals = plsc.load_gather(rows_v.at[s], [bv, dv])
                    plsc.store_scatter(
                        outb_v.at[s], [(pk >> 3) + dt0, pk & 7, bv], vals
                    )
            return carry2

        lax.fori_loop(0, _D // 16, dblock, 0)

        start_write(c, s)
        return carry

    lax.fori_loop(0, _PJ, chunk, 0)

    for s in (0, 1):
        pltpu.make_async_copy(
            outb_v.at[s], out_hbm.at[0, 0, :, wid], wsem.at[s]
        ).wait()


@jax.jit
def _embed_lookup(idx_r, table):
    mesh = plsc.VectorSubcoreMesh(core_axis_name="c", subcore_axis_name="s")
    run = pl.kernel(
        _gather_body,
        out_type=jax.ShapeDtypeStruct(
            (_P, _H, _D // 8, _NW, 8, _NB), jnp.float32
        ),
        mesh=mesh,
        scratch_types=[
            pltpu.VMEM((_P, _H, _NB), jnp.int32),
            pltpu.VMEM((2, _NB, _D), jnp.float32),
            pltpu.VMEM((2, _D // 8, 8, _NB), jnp.float32),
            pltpu.SemaphoreType.DMA((2,)),
            pltpu.SemaphoreType.DMA((2,)),
        ],
        compiler_params=pltpu.CompilerParams(
            use_tc_tiling_on_sc=False, needs_layout_passes=False
        ),
    )
    return run(idx_r, table)


def kernel(idx, embedding_weight):
    idx_r = (
        jnp.transpose(idx, (1, 2, 0))
        .reshape(_P, _H, _NW, _NB)
        .transpose(0, 2, 1, 3)
    )
    out_r = _embed_lookup(idx_r, embedding_weight)
    out = jnp.transpose(out_r, (3, 5, 0, 1, 2, 4)).reshape(
        _BATCH, _P, _H, _D
    )
    return out


# speedup vs baseline: 1.9490x; 1.0867x over previous
import functools

import jax
import jax.numpy as jnp
from jax import lax
from jax.experimental import pallas as pl
from jax.experimental.pallas import tpu as pltpu
from jax.experimental.pallas import tpu_sc as plsc

_D = 64
_P = 50
_H = 2
_BATCH = 4096
_NC = 2
_NS = 16
_NW = _NC * _NS
_NB = _BATCH // _NW
_PJ = _P * _H


def _gather_body(idx_hbm, table_hbm, out_hbm, idx_v, rows_v, outb_v, gsem, wsem):
    wid = lax.axis_index("s") * _NC + lax.axis_index("c")
    pltpu.sync_copy(idx_hbm.at[:, wid], idx_v)

    iota = lax.iota(jnp.int32, 16)
    perms = [(iota + k) & 15 for k in range(16)]

    def start_gather(c, s):
        p = c // _H
        j = c - p * _H
        return pltpu.async_copy(
            table_hbm.at[idx_v.at[p, j]], rows_v.at[s], gsem.at[s]
        )

    def start_write(c, s):
        p = c // _H
        j = c - p * _H
        return pltpu.async_copy(
            outb_v.at[s], out_hbm.at[p, j, :, wid], wsem.at[s]
        )

    start_gather(0, 0)

    def chunk(c, carry):
        s = c & 1

        @pl.when(c >= 2)
        def _():
            start_write_desc = pltpu.make_async_copy(
                outb_v.at[s], out_hbm.at[0, 0, :, wid], wsem.at[s]
            )
            start_write_desc.wait()

        @pl.when(c + 1 < _PJ)
        def _():
            start_gather(c + 1, 1 - s)

        pltpu.make_async_copy(
            table_hbm.at[idx_v.at[0, 0]], rows_v.at[s], gsem.at[s]
        ).wait()

        def dblock(t, carry2):
            d0 = t * 16
            dt0 = t * 2
            for bb in range(_NB // 16):
                bv = iota + (bb * 16)
                for k in range(16):
                    pk = perms[k]
                    dv = pk + d0
                    vals = plsc.load_gather(rows_v.at[s], [bv, dv])
                    plsc.store_scatter(
                        outb_v.at[s], [(pk >> 3) + dt0, pk & 7, bv], vals
                    )
            return carry2

        lax.fori_loop(0, _D // 16, dblock, 0)

        start_write(c, s)
        return carry

    lax.fori_loop(0, _PJ, chunk, 0)

    for s in (0, 1):
        pltpu.make_async_copy(
            outb_v.at[s], out_hbm.at[0, 0, :, wid], wsem.at[s]
        ).wait()


@jax.jit
def _embed_lookup(idx_r, table):
    mesh = plsc.VectorSubcoreMesh(core_axis_name="c", subcore_axis_name="s")
    run = pl.kernel(
        _gather_body,
        out_type=jax.ShapeDtypeStruct(
            (_P, _H, _D // 8, _NW, 8, _NB), jnp.float32
        ),
        mesh=mesh,
        scratch_types=[
            pltpu.VMEM((_P, _H, _NB), jnp.int32),
            pltpu.VMEM((2, _NB, 2 * _D), jnp.float32),
            pltpu.VMEM((2, _D // 8, 8, _NB), jnp.float32),
            pltpu.SemaphoreType.DMA((2,)),
            pltpu.SemaphoreType.DMA((2,)),
        ],
        compiler_params=pltpu.CompilerParams(
            use_tc_tiling_on_sc=False, needs_layout_passes=False
        ),
    )
    return run(idx_r, table)


def kernel(idx, embedding_weight):
    tab_p = jnp.pad(embedding_weight, ((0, 0), (0, 64)))
    idx_r = (
        jnp.transpose(idx, (1, 2, 0))
        .reshape(_P, _H, _NW, _NB)
        .transpose(0, 2, 1, 3)
    )
    out_r = _embed_lookup(idx_r, tab_p)
    out = jnp.transpose(out_r, (3, 5, 0, 1, 2, 4)).reshape(
        _BATCH, _P, _H, _D
    )
    return out
